# baseline (speedup 1.0000x reference)
import functools

import jax
import jax.numpy as jnp
from jax import lax
from jax.experimental import pallas as pl
from jax.experimental.pallas import tpu as pltpu

ALPHA_RELU = 0.15
IN_EPS = 1e-5


def _fused_kernel(x_ref, w_ref, o_ref, *, B, Cin, H, W):
    HW = H * W
    w = w_ref[...]

    col = lax.broadcasted_iota(jnp.int32, (1, HW), 1) % W
    mask_l = (col >= 1).astype(jnp.bfloat16)
    mask_r = (col <= W - 2).astype(jnp.bfloat16)

    BC = B * Cin
    xall = x_ref[...].reshape(BC, HW).astype(jnp.bfloat16)
    zrow = jnp.zeros((BC, W), jnp.bfloat16)
    z1 = jnp.zeros((BC, 1), jnp.bfloat16)
    planes = (
        jnp.concatenate([zrow, xall[:, :HW - W]], axis=1),
        xall,
        jnp.concatenate([xall[:, W:], zrow], axis=1),
    )
    taps = []
    for p in planes:
        taps.append(jnp.concatenate([z1, p[:, :HW - 1]], axis=1) * mask_l)
        taps.append(p)
        taps.append(jnp.concatenate([p[:, 1:], z1], axis=1) * mask_r)

    for b in range(B):
        g = jnp.concatenate([t[b * Cin:(b + 1) * Cin] for t in taps], axis=0)

        acc = jnp.dot(w, g, preferred_element_type=jnp.float32)

        inv_hw = jnp.float32(1.0 / HW)
        mean = jnp.sum(acc, axis=1, keepdims=True) * inv_hw
        ex2 = jnp.sum(acc * acc, axis=1, keepdims=True) * inv_hw
        var = ex2 - mean * mean
        s = lax.rsqrt(var + IN_EPS)
        normed = acc * s - mean * s

        out = jnp.maximum(normed, ALPHA_RELU * normed)
        o_ref[b] = out.astype(o_ref.dtype)


def _conv_layer_call(x_flat, w2, *, Cin, H, W, Cout, kh, kw):
    N = x_flat.shape[0]
    HW = H * W
    B = 4 if N % 4 == 0 else 1
    kern = functools.partial(_fused_kernel, B=B, Cin=Cin, H=H, W=W)

    cost = pl.CostEstimate(
        flops=2 * N * HW * Cin * kh * kw * Cout,
        transcendentals=0,
        bytes_accessed=x_flat.size * 4 + w2.size * 2 + N * Cout * HW * 2,
    )

    return pl.pallas_call(
        kern,
        out_shape=jax.ShapeDtypeStruct((N, Cout, HW), jnp.bfloat16),
        grid=(N // B,),
        in_specs=[
            pl.BlockSpec((B, Cin, HW), lambda n: (n, 0, 0)),
            pl.BlockSpec((Cout, kh * kw * Cin), lambda n: (0, 0)),
        ],
        out_specs=pl.BlockSpec((B, Cout, HW), lambda n: (n, 0, 0)),
        compiler_params=pltpu.CompilerParams(
            dimension_semantics=("parallel",),
            vmem_limit_bytes=64 * 1024 * 1024,
        ),
        cost_estimate=cost,
    )(x_flat, w2)


def kernel(x, weight, bias):
    del bias
    N, Cin, H, W = x.shape
    Cout, Cin_w, kh, kw = weight.shape
    assert Cin_w == Cin and kh == kw == 3
    HW = H * W

    x_flat = x.reshape(N, Cin, HW)
    w2 = jnp.transpose(weight, (0, 2, 3, 1)).reshape(Cout, kh * kw * Cin)
    w2 = w2.astype(jnp.bfloat16)

    out_flat = _conv_layer_call(x_flat, w2, Cin=Cin, H=H, W=W,
                                Cout=Cout, kh=kh, kw=kw)

    return out_flat.astype(jnp.float32).reshape(N, Cout, H, W)


# speedup vs baseline: 1.1475x; 1.1475x over previous
import functools

import jax
import jax.numpy as jnp
from jax import lax
from jax.experimental import pallas as pl
from jax.experimental.pallas import tpu as pltpu

ALPHA_RELU = 0.15
IN_EPS = 1e-5


def _fused_kernel(x_ref, w_ref, o_ref, *, B, Cin, H, W):
    HW = H * W
    w = w_ref[...]

    col = lax.broadcasted_iota(jnp.int32, (1, HW), 1) % W
    mask_l = (col >= 1).astype(jnp.bfloat16)
    mask_r = (col <= W - 2).astype(jnp.bfloat16)

    BC = B * Cin
    xall = x_ref[...].reshape(BC, HW).astype(jnp.bfloat16)
    zrow = jnp.zeros((BC, W), jnp.bfloat16)
    z1 = jnp.zeros((BC, 1), jnp.bfloat16)
    planes = (
        jnp.concatenate([zrow, xall[:, :HW - W]], axis=1),
        xall,
        jnp.concatenate([xall[:, W:], zrow], axis=1),
    )
    taps = []
    for p in planes:
        taps.append(jnp.concatenate([z1, p[:, :HW - 1]], axis=1) * mask_l)
        taps.append(p)
        taps.append(jnp.concatenate([p[:, 1:], z1], axis=1) * mask_r)

    for b in range(B):
        g = jnp.concatenate([t[b * Cin:(b + 1) * Cin] for t in taps], axis=0)

        acc = jnp.dot(w, g, preferred_element_type=jnp.float32)

        inv_hw = jnp.float32(1.0 / HW)
        mean = jnp.sum(acc, axis=1, keepdims=True) * inv_hw
        ex2 = jnp.sum(acc * acc, axis=1, keepdims=True) * inv_hw
        var = ex2 - mean * mean
        s = lax.rsqrt(var + IN_EPS)
        normed = acc * s - mean * s

        out = jnp.maximum(normed, ALPHA_RELU * normed)
        o_ref[b] = out.astype(o_ref.dtype)


def _conv_layer_call(x_flat, w2, *, Cin, H, W, Cout, kh, kw):
    N = x_flat.shape[0]
    HW = H * W
    B = 8 if N % 8 == 0 else (4 if N % 4 == 0 else 1)
    kern = functools.partial(_fused_kernel, B=B, Cin=Cin, H=H, W=W)

    cost = pl.CostEstimate(
        flops=2 * N * HW * Cin * kh * kw * Cout,
        transcendentals=0,
        bytes_accessed=x_flat.size * 4 + w2.size * 2 + N * Cout * HW * 2,
    )

    return pl.pallas_call(
        kern,
        out_shape=jax.ShapeDtypeStruct((N, Cout, HW), jnp.bfloat16),
        grid=(N // B,),
        in_specs=[
            pl.BlockSpec((B, Cin, HW), lambda n: (n, 0, 0)),
            pl.BlockSpec((Cout, kh * kw * Cin), lambda n: (0, 0)),
        ],
        out_specs=pl.BlockSpec((B, Cout, HW), lambda n: (n, 0, 0)),
        compiler_params=pltpu.CompilerParams(
            dimension_semantics=("parallel",),
            vmem_limit_bytes=64 * 1024 * 1024,
        ),
        cost_estimate=cost,
    )(x_flat, w2)


def kernel(x, weight, bias):
    del bias
    N, Cin, H, W = x.shape
    Cout, Cin_w, kh, kw = weight.shape
    assert Cin_w == Cin and kh == kw == 3
    HW = H * W

    x_flat = x.reshape(N, Cin, HW)
    w2 = jnp.transpose(weight, (0, 2, 3, 1)).reshape(Cout, kh * kw * Cin)
    w2 = w2.astype(jnp.bfloat16)

    out_flat = _conv_layer_call(x_flat, w2, Cin=Cin, H=H, W=W,
                                Cout=Cout, kh=kh, kw=kw)

    return out_flat.astype(jnp.float32).reshape(N, Cout, H, W)


# speedup vs baseline: 1.1581x; 1.0092x over previous
import functools

import jax
import jax.numpy as jnp
from jax import lax
from jax.experimental import pallas as pl
from jax.experimental.pallas import tpu as pltpu

ALPHA_RELU = 0.15
IN_EPS = 1e-5


def _fused_kernel(x_ref, w_ref, o_ref, *, B, Cin, H, W):
    HW = H * W
    w = w_ref[...]

    col = lax.broadcasted_iota(jnp.int32, (1, HW), 1) % W
    mask_l = (col >= 1).astype(jnp.bfloat16)
    mask_r = (col <= W - 2).astype(jnp.bfloat16)

    BC = B * Cin
    xall = x_ref[...].reshape(BC, HW).astype(jnp.bfloat16)
    zrow = jnp.zeros((BC, W), jnp.bfloat16)
    z1 = jnp.zeros((BC, 1), jnp.bfloat16)
    lft = jnp.concatenate([z1, xall[:, :HW - 1]], axis=1) * mask_l
    rgt = jnp.concatenate([xall[:, 1:], z1], axis=1) * mask_r

    def _hshift(a, ki):
        if ki == 0:
            return jnp.concatenate([zrow, a[:, :HW - W]], axis=1)
        if ki == 2:
            return jnp.concatenate([a[:, W:], zrow], axis=1)
        return a

    taps = [_hshift(src, ki) for ki in (0, 1, 2) for src in (lft, xall, rgt)]

    for b in range(B):
        g = jnp.concatenate([t[b * Cin:(b + 1) * Cin] for t in taps], axis=0)

        acc = jnp.dot(w, g, preferred_element_type=jnp.float32)

        inv_hw = jnp.float32(1.0 / HW)
        mean = jnp.sum(acc, axis=1, keepdims=True) * inv_hw
        ex2 = jnp.sum(acc * acc, axis=1, keepdims=True) * inv_hw
        var = ex2 - mean * mean
        s = lax.rsqrt(var + IN_EPS)
        normed = acc * s - mean * s

        out = jnp.maximum(normed, ALPHA_RELU * normed)
        o_ref[b] = out.astype(o_ref.dtype)


def _conv_layer_call(x_flat, w2, *, Cin, H, W, Cout, kh, kw):
    N = x_flat.shape[0]
    HW = H * W
    B = 8 if N % 8 == 0 else (4 if N % 4 == 0 else 1)
    kern = functools.partial(_fused_kernel, B=B, Cin=Cin, H=H, W=W)

    cost = pl.CostEstimate(
        flops=2 * N * HW * Cin * kh * kw * Cout,
        transcendentals=0,
        bytes_accessed=x_flat.size * 4 + w2.size * 2 + N * Cout * HW * 2,
    )

    return pl.pallas_call(
        kern,
        out_shape=jax.ShapeDtypeStruct((N, Cout, HW), jnp.bfloat16),
        grid=(N // B,),
        in_specs=[
            pl.BlockSpec((B, Cin, HW), lambda n: (n, 0, 0)),
            pl.BlockSpec((Cout, kh * kw * Cin), lambda n: (0, 0)),
        ],
        out_specs=pl.BlockSpec((B, Cout, HW), lambda n: (n, 0, 0)),
        compiler_params=pltpu.CompilerParams(
            dimension_semantics=("parallel",),
            vmem_limit_bytes=64 * 1024 * 1024,
        ),
        cost_estimate=cost,
    )(x_flat, w2)


def kernel(x, weight, bias):
    del bias
    N, Cin, H, W = x.shape
    Cout, Cin_w, kh, kw = weight.shape
    assert Cin_w == Cin and kh == kw == 3
    HW = H * W

    x_flat = x.reshape(N, Cin, HW)
    w2 = jnp.transpose(weight, (0, 2, 3, 1)).reshape(Cout, kh * kw * Cin)
    w2 = w2.astype(jnp.bfloat16)

    out_flat = _conv_layer_call(x_flat, w2, Cin=Cin, H=H, W=W,
                                Cout=Cout, kh=kh, kw=kw)

    return out_flat.astype(jnp.float32).reshape(N, Cout, H, W)


# speedup vs baseline: 1.1605x; 1.0021x over previous
import functools

import jax
import jax.numpy as jnp
from jax import lax
from jax.experimental import pallas as pl
from jax.experimental.pallas import tpu as pltpu

ALPHA_RELU = 0.15
IN_EPS = 1e-5


def _fused_kernel(x_ref, w_ref, o_ref, g_ref, *, B, Cin, H, W):
    HW = H * W
    w = w_ref[...]

    col = lax.broadcasted_iota(jnp.int32, (1, HW), 1) % W
    mask_l = (col >= 1).astype(jnp.bfloat16)
    mask_r = (col <= W - 2).astype(jnp.bfloat16)

    BC = B * Cin
    xall = x_ref[...].reshape(BC, HW).astype(jnp.bfloat16)
    zrow = jnp.zeros((BC, W), jnp.bfloat16)
    z1 = jnp.zeros((BC, 1), jnp.bfloat16)
    lft = jnp.concatenate([z1, xall[:, :HW - 1]], axis=1) * mask_l
    rgt = jnp.concatenate([xall[:, 1:], z1], axis=1) * mask_r

    def _hshift(a, ki):
        if ki == 0:
            return jnp.concatenate([zrow, a[:, :HW - W]], axis=1)
        if ki == 2:
            return jnp.concatenate([a[:, W:], zrow], axis=1)
        return a

    t = 0
    for ki in (0, 1, 2):
        for srcp in (lft, xall, rgt):
            g_ref[:, t * Cin:(t + 1) * Cin, :] = _hshift(srcp, ki).reshape(B, Cin, HW)
            t += 1

    for b in range(B):
        acc = jnp.dot(w, g_ref[b], preferred_element_type=jnp.float32)

        inv_hw = jnp.float32(1.0 / HW)
        mean = jnp.sum(acc, axis=1, keepdims=True) * inv_hw
        ex2 = jnp.sum(acc * acc, axis=1, keepdims=True) * inv_hw
        var = ex2 - mean * mean
        s = lax.rsqrt(var + IN_EPS)
        normed = acc * s - mean * s

        out = jnp.maximum(normed, ALPHA_RELU * normed)
        o_ref[b] = out.astype(o_ref.dtype)


def _conv_layer_call(x_flat, w2, *, Cin, H, W, Cout, kh, kw):
    N = x_flat.shape[0]
    HW = H * W
    B = 8 if N % 8 == 0 else (4 if N % 4 == 0 else 1)
    kern = functools.partial(_fused_kernel, B=B, Cin=Cin, H=H, W=W)

    cost = pl.CostEstimate(
        flops=2 * N * HW * Cin * kh * kw * Cout,
        transcendentals=0,
        bytes_accessed=x_flat.size * 4 + w2.size * 2 + N * Cout * HW * 2,
    )

    return pl.pallas_call(
        kern,
        out_shape=jax.ShapeDtypeStruct((N, Cout, HW), jnp.bfloat16),
        grid=(N // B,),
        in_specs=[
            pl.BlockSpec((B, Cin, HW), lambda n: (n, 0, 0)),
            pl.BlockSpec((Cout, kh * kw * Cin), lambda n: (0, 0)),
        ],
        out_specs=pl.BlockSpec((B, Cout, HW), lambda n: (n, 0, 0)),
        scratch_shapes=[pltpu.VMEM((B, 3 * kh * Cin, HW), jnp.bfloat16)],
        compiler_params=pltpu.CompilerParams(
            dimension_semantics=("parallel",),
            vmem_limit_bytes=64 * 1024 * 1024,
        ),
        cost_estimate=cost,
    )(x_flat, w2)


def kernel(x, weight, bias):
    del bias
    N, Cin, H, W = x.shape
    Cout, Cin_w, kh, kw = weight.shape
    assert Cin_w == Cin and kh == kw == 3
    HW = H * W

    x_flat = x.reshape(N, Cin, HW)
    w2 = jnp.transpose(weight, (0, 2, 3, 1)).reshape(Cout, kh * kw * Cin)
    w2 = w2.astype(jnp.bfloat16)

    out_flat = _conv_layer_call(x_flat, w2, Cin=Cin, H=H, W=W,
                                Cout=Cout, kh=kh, kw=kw)

    return out_flat.astype(jnp.float32).reshape(N, Cout, H, W)


# speedup vs baseline: 1.1664x; 1.0051x over previous
import functools

import jax
import jax.numpy as jnp
from jax import lax
from jax.experimental import pallas as pl
from jax.experimental.pallas import tpu as pltpu

ALPHA_RELU = 0.15
IN_EPS = 1e-5


def _fused_kernel(x_ref, w_ref, o_ref, g_ref, *, B, Cin, H, W):
    HW = H * W
    w = w_ref[...]

    col = lax.broadcasted_iota(jnp.int32, (1, HW), 1) % W
    mask_l = (col >= 1).astype(jnp.bfloat16)
    mask_r = (col <= W - 2).astype(jnp.bfloat16)

    BC = B * Cin
    xall = x_ref[...].reshape(BC, HW).astype(jnp.bfloat16)
    zrow = jnp.zeros((BC, W), jnp.bfloat16)
    z1 = jnp.zeros((BC, 1), jnp.bfloat16)
    lft = jnp.concatenate([z1, xall[:, :HW - 1]], axis=1) * mask_l
    rgt = jnp.concatenate([xall[:, 1:], z1], axis=1) * mask_r

    def _hshift(a, ki):
        if ki == 0:
            return jnp.concatenate([zrow, a[:, :HW - W]], axis=1)
        if ki == 2:
            return jnp.concatenate([a[:, W:], zrow], axis=1)
        return a

    t = 0
    for ki in (0, 1, 2):
        for srcp in (lft, xall, rgt):
            g_ref[:, t * Cin:(t + 1) * Cin, :] = _hshift(srcp, ki).reshape(B, Cin, HW)
            t += 1

    for b2 in range(B // 2):
        gpair = jnp.concatenate([g_ref[2 * b2], g_ref[2 * b2 + 1]], axis=1)
        accp = jnp.dot(w, gpair, preferred_element_type=jnp.float32)

        inv_hw = jnp.float32(1.0 / HW)
        for half in range(2):
            acc = accp[:, half * HW:(half + 1) * HW]
            mean = jnp.sum(acc, axis=1, keepdims=True) * inv_hw
            ex2 = jnp.sum(acc * acc, axis=1, keepdims=True) * inv_hw
            var = ex2 - mean * mean
            s = lax.rsqrt(var + IN_EPS)
            normed = acc * s - mean * s

            out = jnp.maximum(normed, ALPHA_RELU * normed)
            o_ref[2 * b2 + half] = out.astype(o_ref.dtype)


def _conv_layer_call(x_flat, w2, *, Cin, H, W, Cout, kh, kw):
    N = x_flat.shape[0]
    HW = H * W
    B = 8 if N % 8 == 0 else (4 if N % 4 == 0 else 1)
    kern = functools.partial(_fused_kernel, B=B, Cin=Cin, H=H, W=W)

    cost = pl.CostEstimate(
        flops=2 * N * HW * Cin * kh * kw * Cout,
        transcendentals=0,
        bytes_accessed=x_flat.size * 4 + w2.size * 2 + N * Cout * HW * 2,
    )

    return pl.pallas_call(
        kern,
        out_shape=jax.ShapeDtypeStruct((N, Cout, HW), jnp.bfloat16),
        grid=(N // B,),
        in_specs=[
            pl.BlockSpec((B, Cin, HW), lambda n: (n, 0, 0)),
            pl.BlockSpec((Cout, kh * kw * Cin), lambda n: (0, 0)),
        ],
        out_specs=pl.BlockSpec((B, Cout, HW), lambda n: (n, 0, 0)),
        scratch_shapes=[pltpu.VMEM((B, 3 * kh * Cin, HW), jnp.bfloat16)],
        compiler_params=pltpu.CompilerParams(
            dimension_semantics=("parallel",),
            vmem_limit_bytes=64 * 1024 * 1024,
        ),
        cost_estimate=cost,
    )(x_flat, w2)


def kernel(x, weight, bias):
    del bias
    N, Cin, H, W = x.shape
    Cout, Cin_w, kh, kw = weight.shape
    assert Cin_w == Cin and kh == kw == 3
    HW = H * W

    x_flat = x.reshape(N, Cin, HW)
    w2 = jnp.transpose(weight, (0, 2, 3, 1)).reshape(Cout, kh * kw * Cin)
    w2 = w2.astype(jnp.bfloat16)

    out_flat = _conv_layer_call(x_flat, w2, Cin=Cin, H=H, W=W,
                                Cout=Cout, kh=kh, kw=kw)

    return out_flat.astype(jnp.float32).reshape(N, Cout, H, W)
